# baseline (speedup 1.0000x reference)
import functools

import jax
import jax.numpy as jnp
from jax import lax
from jax.experimental import pallas as pl
from jax.experimental.pallas import tpu as pltpu
from jax.experimental.pallas import tpu_sc as plsc

_B = 16384
_B4 = _B // 4
_D = 32
_RD = 32
_NREL = 64
_NW = 32
_BPW = _B // _NW
_NG = 8
_RG = _NREL // _NG
_KP = _RG * _D
_NP = _NG * _RD
_BLK4 = 256
_BLK = 4 * _BLK4
_NB = _B // _BLK
_CF_LAMBDA = 1e-05
_KG_LAMBDA = 1e-05

_N_ENT = 1000000
_N_USR = 1000000
_N_ITM = 100000
_UCH = 320
_UNCH = _N_USR // _UCH
_ICH = 160
_INCH = _N_ITM // _ICH


def _tc_pack(table, n, blkr):
    def body(in_ref, out_ref):
        y = in_ref[...].reshape(blkr // 4, 4, 32)
        out_ref[...] = jnp.concatenate([y[:, q, :] for q in range(4)], axis=1)

    return pl.pallas_call(
        body, grid=(n // blkr,),
        in_specs=[pl.BlockSpec((blkr, 32), lambda i: (i, 0))],
        out_specs=pl.BlockSpec((blkr // 4, 128), lambda i: (i, 0)),
        out_shape=jax.ShapeDtypeStruct((n // 4, 128), jnp.float32))(table)


def _sc_pack_one(tbl, out, ub, ob, sems, w, nch, ch, npack):
    nw2 = (nch + 2 * _NW - 1) // (2 * _NW)
    copies = [None, None]
    for par in (0, 1):
        cid0 = w + _NW * par
        @pl.when(cid0 < nch)
        def _(par=par, cid0=cid0):
            pltpu.async_copy(tbl.at[pl.ds(pl.multiple_of(cid0 * ch, 8), ch)],
                             ub[par].at[pl.ds(0, ch)], sems[par])

    def body(c2, _):
        for par in (0, 1):
            c = 2 * c2 + par
            cid = w + _NW * c

            @pl.when(cid < nch)
            def _(par=par, c=c, cid=cid):
                pltpu.make_async_copy(
                    tbl.at[pl.ds(pl.multiple_of(cid * ch, 8), ch)],
                    ub[par].at[pl.ds(0, ch)], sems[par]).wait()

                def pack(m, _):
                    for qq in range(4):
                        j = 4 * m + qq
                        ob[m, pl.ds(qq * 32, 16)] = ub[par][j, 0:16]
                        ob[m, pl.ds(qq * 32 + 16, 16)] = ub[par][j, pl.ds(16, 16)]
                    return 0

                lax.fori_loop(0, npack, pack, 0, unroll=4)
                pltpu.sync_copy(
                    ob.at[pl.ds(0, npack)],
                    out.at[pl.ds(pl.multiple_of(cid * npack, 8), npack)])
                nxt = cid + 2 * _NW

                @pl.when(nxt < nch)
                def _():
                    pltpu.async_copy(
                        tbl.at[pl.ds(pl.multiple_of(nxt * ch, 8), ch)],
                        ub[par].at[pl.ds(0, ch)], sems[par])
        return 0

    lax.fori_loop(0, nw2, body, 0)


def _sc_pack_kernel(user, item, up, ip, ub0, ub1, ob, s0, s1):
    w = lax.axis_index("s") * 2 + lax.axis_index("c")
    _sc_pack_one(user, up, (ub0, ub1), ob, (s0, s1), w, _UNCH, _UCH, _UCH // 4)
    _sc_pack_one(item, ip, (ub0, ub1), ob, (s0, s1), w, _INCH, _ICH, _ICH // 4)


def _sc_pack(user_embed, item_embed):
    mesh = plsc.VectorSubcoreMesh(core_axis_name="c", subcore_axis_name="s")
    k = functools.partial(
        pl.kernel,
        mesh=mesh,
        compiler_params=pltpu.CompilerParams(needs_layout_passes=False),
        out_type=(jax.ShapeDtypeStruct((_N_USR // 4, 128), jnp.float32),
                  jax.ShapeDtypeStruct((_N_ITM // 4, 128), jnp.float32)),
        scratch_types=[
            pltpu.VMEM((_UCH, _D), jnp.float32),
            pltpu.VMEM((_UCH, _D), jnp.float32),
            pltpu.VMEM((_UCH // 4, 128), jnp.float32),
            pltpu.SemaphoreType.DMA,
            pltpu.SemaphoreType.DMA,
        ],
    )(_sc_pack_kernel)
    return k(user_embed, item_embed)


_GCH = 128


def _sc_gather_kernel(ep, up, ipk, uid_h, ipi_h, ini_h, h_h, pt_h, nt_h, out,
                      iv, tv0, tv1, tb0, tb1, ob, s0, s1, so):
    w = lax.axis_index("s") * 2 + lax.axis_index("c")
    base = pl.multiple_of(w * _BPW, 8)
    sl = pl.ds(base, _BPW)
    tasks = [(ep, h_h), (ep, pt_h), (ep, nt_h), (up, uid_h),
             (ipk, ipi_h), (ipk, ini_h), (ep, ipi_h), (ep, ini_h)]
    tvs = (tv0, tv1)
    tbs = (tb0, tb1)
    sems = (s0, s1)
    owrite = [None]
    for g, (tbl, idx_h) in enumerate(tasks):
        pltpu.sync_copy(idx_h.at[sl], iv)
        for v in range(_GCH // 16):
            tv0[pl.ds(v * 16, 16)] = lax.shift_right_logical(
                iv[pl.ds(v * 16, 16)], 2)
        pltpu.async_copy(tbl.at[tv0], tb0, s0)
        for c in range(_BPW // _GCH):
            par = c % 2
            npar = (c + 1) % 2
            if c + 1 < _BPW // _GCH:
                off = (c + 1) * _GCH
                for v in range(_GCH // 16):
                    tvs[npar][pl.ds(v * 16, 16)] = lax.shift_right_logical(
                        iv[pl.ds(off + v * 16, 16)], 2)
            pltpu.make_async_copy(tbl.at[tvs[par]], tbs[par], sems[par]).wait()
            if c + 1 < _BPW // _GCH:
                pltpu.async_copy(tbl.at[tvs[npar]], tbs[npar], sems[npar])

            ii = lax.iota(jnp.int32, 16)

            def extract(m, _, c=c, par=par):
                for qq in range(4):
                    j = 4 * m + qq
                    jb = jnp.zeros((16,), jnp.int32) + j
                    idxb = plsc.load_gather(iv, [jb + (c * _GCH)])
                    q32 = lax.bitwise_and(idxb, 3) * 32
                    colv = q32 + ii
                    v0 = plsc.load_gather(tbs[par], [jb, colv])
                    v1 = plsc.load_gather(tbs[par], [jb, colv + 16])
                    ob[c * (_GCH // 4) + m, pl.ds(qq * 32, 16)] = v0
                    ob[c * (_GCH // 4) + m, pl.ds(qq * 32 + 16, 16)] = v1
                return 0

            lax.fori_loop(0, _GCH // 4, extract, 0, unroll=2)
        if owrite[0] is not None:
            owrite[0].wait()
        owrite[0] = pltpu.async_copy(
            ob, out.at[g, pl.ds(pl.multiple_of(w * (_BPW // 4), 8),
                                _BPW // 4)], so)
    owrite[0].wait()


def _sc_gather(ep, up, ipk, uid, ipi, ini, h, pt, nt):
    mesh = plsc.VectorSubcoreMesh(core_axis_name="c", subcore_axis_name="s")
    k = functools.partial(
        pl.kernel,
        mesh=mesh,
        compiler_params=pltpu.CompilerParams(needs_layout_passes=False),
        out_type=jax.ShapeDtypeStruct((8, _B4, 128), jnp.float32),
        scratch_types=[
            pltpu.VMEM((_BPW,), jnp.int32),
            pltpu.VMEM((_GCH,), jnp.int32),
            pltpu.VMEM((_GCH,), jnp.int32),
            pltpu.VMEM((_GCH, 128), jnp.float32),
            pltpu.VMEM((_GCH, 128), jnp.float32),
            pltpu.VMEM((_BPW // 4, 128), jnp.float32),
            pltpu.SemaphoreType.DMA,
            pltpu.SemaphoreType.DMA,
            pltpu.SemaphoreType.DMA,
        ],
    )(_sc_gather_kernel)
    return k(ep, up, ipk, uid, ipi, ini, h, pt, nt)


def _logsig(x):
    return jnp.minimum(x, 0.0) - jnp.log1p(jnp.exp(-jnp.abs(x)))


def _nrm(v):
    n = jnp.sqrt(jnp.sum(v * v, axis=1, keepdims=True))
    return v / jnp.maximum(n, 1e-12)


def _l2h(v):
    return 0.5 * jnp.sum(v * v)


def _deint(g_ref, a):
    arr = g_ref[a]
    return [arr[:, q * 32:(q + 1) * 32] for q in range(4)]


def _tc_body(g_ref, r_ref, w_ref, re_ref, out_ref, acc_ref):
    i = pl.program_id(0)

    @pl.when(i == 0)
    def _():
        for k in range(9):
            acc_ref[k] = 0.0

    rcat = jnp.concatenate(
        [r_ref[:, q:q + 1] for q in range(4)], axis=0)
    rl = lax.bitwise_and(rcat, _RG - 1)
    gg = lax.shift_right_logical(rcat, 3)

    h_l = _deint(g_ref, 0)
    pt_l = _deint(g_ref, 1)
    nt_l = _deint(g_ref, 2)

    x3 = jnp.concatenate(h_l + pt_l + nt_l, axis=0)
    xt = pltpu.repeat(x3, _RG, axis=1)
    col = lax.broadcasted_iota(jnp.int32, (3 * _BLK, _KP), 1)
    rlcol = lax.shift_right_logical(col, 5)
    rl3 = jnp.concatenate([rl, rl, rl], axis=0)
    a = jnp.where(rlcol == rl3, xt, 0.0)
    rm_big = jnp.dot(a, w_ref[...], preferred_element_type=jnp.float32)
    gg3 = jnp.concatenate([gg, gg, gg], axis=0)
    rm3 = jnp.zeros((3 * _BLK, _RD), jnp.float32)
    for g in range(_NG):
        rm3 = rm3 + jnp.where(gg3 == g, rm_big[:, g * _RD:(g + 1) * _RD], 0.0)
    rmh = rm3[0:_BLK]
    rmpt = rm3[_BLK:2 * _BLK]
    rmnt = rm3[2 * _BLK:]

    oh = (lax.broadcasted_iota(jnp.int32, (_BLK, _NREL), 1) == rcat)
    r_e = jnp.dot(oh.astype(jnp.float32), re_ref[...],
                  preferred_element_type=jnp.float32)

    rmh_n = _nrm(rmh)
    re_n = _nrm(r_e)
    rmpt_n = _nrm(rmpt)
    rmnt_n = _nrm(rmnt)
    pos_sc = jnp.sum((rmh_n + re_n - rmpt_n) ** 2, axis=1, keepdims=True)
    neg_sc = jnp.sum((rmh_n + re_n - rmnt_n) ** 2, axis=1, keepdims=True)
    kg_ls = jnp.sum(-_logsig(neg_sc - pos_sc))

    u_e = jnp.concatenate(_deint(g_ref, 3), axis=0)
    ip_e = jnp.concatenate(_deint(g_ref, 4), axis=0)
    in_e = jnp.concatenate(_deint(g_ref, 5), axis=0)
    ipk_e = jnp.concatenate(_deint(g_ref, 6), axis=0)
    ink_e = jnp.concatenate(_deint(g_ref, 7), axis=0)
    ip_cf = ip_e + ipk_e
    in_cf = in_e + ink_e
    pos_s = jnp.sum(u_e * ip_cf, axis=1, keepdims=True)
    neg_s = jnp.sum(u_e * in_cf, axis=1, keepdims=True)
    cf_ls = jnp.sum(-_logsig(pos_s - neg_s))

    acc_ref[0] = acc_ref[0] + kg_ls
    acc_ref[1] = acc_ref[1] + _l2h(rmh_n)
    acc_ref[2] = acc_ref[2] + _l2h(re_n)
    acc_ref[3] = acc_ref[3] + _l2h(rmpt_n)
    acc_ref[4] = acc_ref[4] + _l2h(rmnt_n)
    acc_ref[5] = acc_ref[5] + cf_ls
    acc_ref[6] = acc_ref[6] + _l2h(u_e)
    acc_ref[7] = acc_ref[7] + _l2h(ip_cf)
    acc_ref[8] = acc_ref[8] + _l2h(in_cf)

    @pl.when(i == _NB - 1)
    def _():
        bf = jnp.float32(_B)
        kg_total = acc_ref[0] / bf + _KG_LAMBDA * (
            (acc_ref[1] + acc_ref[2] + acc_ref[3] + acc_ref[4]) / bf)
        cf_total = acc_ref[5] / bf + _CF_LAMBDA * (
            (acc_ref[6] + acc_ref[7] + acc_ref[8]) / bf)
        out_ref[0, 0] = kg_total + cf_total


def _tc_loss(gath, r4, w_big, rel_emb):
    return pl.pallas_call(
        _tc_body,
        grid=(_NB,),
        in_specs=[
            pl.BlockSpec((8, _BLK4, 128), lambda i: (0, i, 0)),
            pl.BlockSpec((_BLK4, 4), lambda i: (i, 0)),
            pl.BlockSpec((_KP, _NP), lambda i: (0, 0)),
            pl.BlockSpec((_NREL, _RD), lambda i: (0, 0)),
        ],
        out_specs=pl.BlockSpec((1, 1), lambda i: (0, 0),
                               memory_space=pltpu.SMEM),
        out_shape=jax.ShapeDtypeStruct((1, 1), jnp.float32),
        scratch_shapes=[pltpu.SMEM((16,), jnp.float32)],
    )(gath, r4, w_big, rel_emb)


def kernel(user_ids, item_pos_ids, item_neg_ids, h, r, pos_t, neg_t,
           user_embed, item_embed, entity_embed, relation_embed, trans_M):
    uid = user_ids.astype(jnp.int32)
    ipi = item_pos_ids.astype(jnp.int32)
    ini = item_neg_ids.astype(jnp.int32)
    h32 = h.astype(jnp.int32)
    pt = pos_t.astype(jnp.int32)
    nt = neg_t.astype(jnp.int32)
    ep = _tc_pack(entity_embed, _N_ENT, 8000)
    up, ipk = _sc_pack(user_embed, item_embed)
    gath = _sc_gather(ep, up, ipk, uid, ipi, ini, h32, pt, nt)
    w_big = trans_M.reshape(_NG, _RG, _D, _RD).transpose(1, 2, 0, 3)
    w_big = w_big.reshape(_KP, _NP)
    r4 = r.astype(jnp.int32).reshape(_B4, 4)
    out = _tc_loss(gath, r4, w_big, relation_embed)
    return out.reshape(())


# speedup vs baseline: 1.0247x; 1.0247x over previous
import functools

import jax
import jax.numpy as jnp
from jax import lax
from jax.experimental import pallas as pl
from jax.experimental.pallas import tpu as pltpu
from jax.experimental.pallas import tpu_sc as plsc

_B = 16384
_B4 = _B // 4
_D = 32
_RD = 32
_NREL = 64
_NW = 32
_BPW = _B // _NW
_NG = 8
_RG = _NREL // _NG
_KP = _RG * _D
_NP = _NG * _RD
_BLK4 = 256
_BLK = 4 * _BLK4
_NB = _B // _BLK
_CF_LAMBDA = 1e-05
_KG_LAMBDA = 1e-05

_N_ENT = 1000000
_N_USR = 1000000
_N_ITM = 100000
_UCH = 320
_UNCH = _N_USR // _UCH
_ICH = 160
_INCH = _N_ITM // _ICH


def _tc_pack(table, n, blkr):
    def body(in_ref, out_ref):
        y = in_ref[...].reshape(blkr // 4, 4, 32)
        out_ref[...] = jnp.concatenate([y[:, q, :] for q in range(4)], axis=1)

    return pl.pallas_call(
        body, grid=(n // blkr,),
        in_specs=[pl.BlockSpec((blkr, 32), lambda i: (i, 0))],
        out_specs=pl.BlockSpec((blkr // 4, 128), lambda i: (i, 0)),
        out_shape=jax.ShapeDtypeStruct((n // 4, 128), jnp.float32))(table)


def _sc_pack_one(tbl, out, ub, ob, sems, osems, w, nch, ch, npack):
    nw2 = (nch + 2 * _NW - 1) // (2 * _NW)
    for par in (0, 1):
        cid0 = w + _NW * par
        @pl.when(cid0 < nch)
        def _(par=par, cid0=cid0):
            pltpu.async_copy(tbl.at[pl.ds(pl.multiple_of(cid0 * ch, 8), ch)],
                             ub[par].at[pl.ds(0, ch)], sems[par])

    def body(c2, _):
        for par in (0, 1):
            c = 2 * c2 + par
            cid = w + _NW * c

            @pl.when(cid < nch)
            def _(par=par, c=c, cid=cid):
                pltpu.make_async_copy(
                    tbl.at[pl.ds(pl.multiple_of(cid * ch, 8), ch)],
                    ub[par].at[pl.ds(0, ch)], sems[par]).wait()

                prev = cid - 2 * _NW

                @pl.when(prev >= 0)
                def _():
                    pltpu.make_async_copy(
                        ob[par].at[pl.ds(0, npack)],
                        out.at[pl.ds(pl.multiple_of(prev * npack, 8), npack)],
                        osems[par]).wait()

                def pack(m, _):
                    for qq in range(4):
                        j = 4 * m + qq
                        ob[par][m, pl.ds(qq * 32, 16)] = ub[par][j, 0:16]
                        ob[par][m, pl.ds(qq * 32 + 16, 16)] = (
                            ub[par][j, pl.ds(16, 16)])
                    return 0

                lax.fori_loop(0, npack, pack, 0, unroll=4)
                nxt = cid + 2 * _NW

                @pl.when(nxt < nch)
                def _():
                    pltpu.async_copy(
                        tbl.at[pl.ds(pl.multiple_of(nxt * ch, 8), ch)],
                        ub[par].at[pl.ds(0, ch)], sems[par])
                pltpu.async_copy(
                    ob[par].at[pl.ds(0, npack)],
                    out.at[pl.ds(pl.multiple_of(cid * npack, 8), npack)],
                    osems[par])
        return 0

    lax.fori_loop(0, nw2, body, 0)
    ncw = (nch - w + _NW - 1) // _NW
    for par in (0, 1):
        @pl.when(ncw > par)
        def _(par=par):
            lastc = ((ncw - 1 - par) // 2) * 2 + par
            lastcid = w + _NW * lastc
            pltpu.make_async_copy(
                ob[par].at[pl.ds(0, npack)],
                out.at[pl.ds(pl.multiple_of(lastcid * npack, 8), npack)],
                osems[par]).wait()


def _sc_pack_kernel(user, item, up, ip, ub0, ub1, ob0, ob1, s0, s1, o0, o1):
    w = lax.axis_index("s") * 2 + lax.axis_index("c")
    _sc_pack_one(user, up, (ub0, ub1), (ob0, ob1), (s0, s1), (o0, o1),
                 w, _UNCH, _UCH, _UCH // 4)
    _sc_pack_one(item, ip, (ub0, ub1), (ob0, ob1), (s0, s1), (o0, o1),
                 w, _INCH, _ICH, _ICH // 4)


def _sc_pack(user_embed, item_embed):
    mesh = plsc.VectorSubcoreMesh(core_axis_name="c", subcore_axis_name="s")
    k = functools.partial(
        pl.kernel,
        mesh=mesh,
        compiler_params=pltpu.CompilerParams(
            needs_layout_passes=False, use_tc_tiling_on_sc=True),
        out_type=(jax.ShapeDtypeStruct((_N_USR // 4, 128), jnp.float32),
                  jax.ShapeDtypeStruct((_N_ITM // 4, 128), jnp.float32)),
        scratch_types=[
            pltpu.VMEM((_UCH, _D), jnp.float32),
            pltpu.VMEM((_UCH, _D), jnp.float32),
            pltpu.VMEM((_UCH // 4, 128), jnp.float32),
            pltpu.VMEM((_UCH // 4, 128), jnp.float32),
            pltpu.SemaphoreType.DMA,
            pltpu.SemaphoreType.DMA,
            pltpu.SemaphoreType.DMA,
            pltpu.SemaphoreType.DMA,
        ],
    )(_sc_pack_kernel)
    return k(user_embed, item_embed)


_GCH = 128


def _sc_gather_kernel(ep, up, ipk, uid_h, ipi_h, ini_h, h_h, pt_h, nt_h, out,
                      iv, tv0, tv1, tb0, tb1, ob, s0, s1, so):
    w = lax.axis_index("s") * 2 + lax.axis_index("c")
    base = pl.multiple_of(w * _BPW, 8)
    sl = pl.ds(base, _BPW)
    tasks = [(ep, h_h), (ep, pt_h), (ep, nt_h), (up, uid_h),
             (ipk, ipi_h), (ipk, ini_h), (ep, ipi_h), (ep, ini_h)]
    tvs = (tv0, tv1)
    tbs = (tb0, tb1)
    sems = (s0, s1)
    owrite = [None]
    for g, (tbl, idx_h) in enumerate(tasks):
        pltpu.sync_copy(idx_h.at[sl], iv)
        for v in range(_GCH // 16):
            tv0[pl.ds(v * 16, 16)] = lax.shift_right_logical(
                iv[pl.ds(v * 16, 16)], 2)
        pltpu.async_copy(tbl.at[tv0], tb0, s0)
        for c in range(_BPW // _GCH):
            par = c % 2
            npar = (c + 1) % 2
            if c + 1 < _BPW // _GCH:
                off = (c + 1) * _GCH
                for v in range(_GCH // 16):
                    tvs[npar][pl.ds(v * 16, 16)] = lax.shift_right_logical(
                        iv[pl.ds(off + v * 16, 16)], 2)
            pltpu.make_async_copy(tbl.at[tvs[par]], tbs[par], sems[par]).wait()
            if c + 1 < _BPW // _GCH:
                pltpu.async_copy(tbl.at[tvs[npar]], tbs[npar], sems[npar])

            ii = lax.iota(jnp.int32, 16)

            def extract(m, _, c=c, par=par):
                for qq in range(4):
                    j = 4 * m + qq
                    jb = jnp.zeros((16,), jnp.int32) + j
                    idxb = plsc.load_gather(iv, [jb + (c * _GCH)])
                    q32 = lax.bitwise_and(idxb, 3) * 32
                    colv = q32 + ii
                    v0 = plsc.load_gather(tbs[par], [jb, colv])
                    v1 = plsc.load_gather(tbs[par], [jb, colv + 16])
                    ob[c * (_GCH // 4) + m, pl.ds(qq * 32, 16)] = v0
                    ob[c * (_GCH // 4) + m, pl.ds(qq * 32 + 16, 16)] = v1
                return 0

            lax.fori_loop(0, _GCH // 4, extract, 0, unroll=2)
        if owrite[0] is not None:
            owrite[0].wait()
        owrite[0] = pltpu.async_copy(
            ob, out.at[g, pl.ds(pl.multiple_of(w * (_BPW // 4), 8),
                                _BPW // 4)], so)
    owrite[0].wait()


def _sc_gather(ep, up, ipk, uid, ipi, ini, h, pt, nt):
    mesh = plsc.VectorSubcoreMesh(core_axis_name="c", subcore_axis_name="s")
    k = functools.partial(
        pl.kernel,
        mesh=mesh,
        compiler_params=pltpu.CompilerParams(
            needs_layout_passes=False, use_tc_tiling_on_sc=True),
        out_type=jax.ShapeDtypeStruct((8, _B4, 128), jnp.float32),
        scratch_types=[
            pltpu.VMEM((_BPW,), jnp.int32),
            pltpu.VMEM((_GCH,), jnp.int32),
            pltpu.VMEM((_GCH,), jnp.int32),
            pltpu.VMEM((_GCH, 128), jnp.float32),
            pltpu.VMEM((_GCH, 128), jnp.float32),
            pltpu.VMEM((_BPW // 4, 128), jnp.float32),
            pltpu.SemaphoreType.DMA,
            pltpu.SemaphoreType.DMA,
            pltpu.SemaphoreType.DMA,
        ],
    )(_sc_gather_kernel)
    return k(ep, up, ipk, uid, ipi, ini, h, pt, nt)


def _logsig(x):
    return jnp.minimum(x, 0.0) - jnp.log1p(jnp.exp(-jnp.abs(x)))


def _nrm(v):
    n = jnp.sqrt(jnp.sum(v * v, axis=1, keepdims=True))
    return v / jnp.maximum(n, 1e-12)


def _l2h(v):
    return 0.5 * jnp.sum(v * v)


def _deint(g_ref, a):
    arr = g_ref[a]
    return [arr[:, q * 32:(q + 1) * 32] for q in range(4)]


def _tc_body(g_ref, r_ref, w_ref, re_ref, out_ref, acc_ref):
    i = pl.program_id(0)

    @pl.when(i == 0)
    def _():
        for k in range(9):
            acc_ref[k] = 0.0

    rcat = jnp.concatenate(
        [r_ref[:, q:q + 1] for q in range(4)], axis=0)
    rl = lax.bitwise_and(rcat, _RG - 1)
    gg = lax.shift_right_logical(rcat, 3)

    h_l = _deint(g_ref, 0)
    pt_l = _deint(g_ref, 1)
    nt_l = _deint(g_ref, 2)

    x3 = jnp.concatenate(h_l + pt_l + nt_l, axis=0)
    xt = pltpu.repeat(x3, _RG, axis=1)
    col = lax.broadcasted_iota(jnp.int32, (3 * _BLK, _KP), 1)
    rlcol = lax.shift_right_logical(col, 5)
    rl3 = jnp.concatenate([rl, rl, rl], axis=0)
    a = jnp.where(rlcol == rl3, xt, 0.0)
    rm_big = jnp.dot(a, w_ref[...], preferred_element_type=jnp.float32)
    gg3 = jnp.concatenate([gg, gg, gg], axis=0)
    rm3 = jnp.zeros((3 * _BLK, _RD), jnp.float32)
    for g in range(_NG):
        rm3 = rm3 + jnp.where(gg3 == g, rm_big[:, g * _RD:(g + 1) * _RD], 0.0)
    rmh = rm3[0:_BLK]
    rmpt = rm3[_BLK:2 * _BLK]
    rmnt = rm3[2 * _BLK:]

    oh = (lax.broadcasted_iota(jnp.int32, (_BLK, _NREL), 1) == rcat)
    r_e = jnp.dot(oh.astype(jnp.float32), re_ref[...],
                  preferred_element_type=jnp.float32)

    rmh_n = _nrm(rmh)
    re_n = _nrm(r_e)
    rmpt_n = _nrm(rmpt)
    rmnt_n = _nrm(rmnt)
    pos_sc = jnp.sum((rmh_n + re_n - rmpt_n) ** 2, axis=1, keepdims=True)
    neg_sc = jnp.sum((rmh_n + re_n - rmnt_n) ** 2, axis=1, keepdims=True)
    kg_ls = jnp.sum(-_logsig(neg_sc - pos_sc))

    u_e = jnp.concatenate(_deint(g_ref, 3), axis=0)
    ip_e = jnp.concatenate(_deint(g_ref, 4), axis=0)
    in_e = jnp.concatenate(_deint(g_ref, 5), axis=0)
    ipk_e = jnp.concatenate(_deint(g_ref, 6), axis=0)
    ink_e = jnp.concatenate(_deint(g_ref, 7), axis=0)
    ip_cf = ip_e + ipk_e
    in_cf = in_e + ink_e
    pos_s = jnp.sum(u_e * ip_cf, axis=1, keepdims=True)
    neg_s = jnp.sum(u_e * in_cf, axis=1, keepdims=True)
    cf_ls = jnp.sum(-_logsig(pos_s - neg_s))

    acc_ref[0] = acc_ref[0] + kg_ls
    acc_ref[1] = acc_ref[1] + _l2h(rmh_n)
    acc_ref[2] = acc_ref[2] + _l2h(re_n)
    acc_ref[3] = acc_ref[3] + _l2h(rmpt_n)
    acc_ref[4] = acc_ref[4] + _l2h(rmnt_n)
    acc_ref[5] = acc_ref[5] + cf_ls
    acc_ref[6] = acc_ref[6] + _l2h(u_e)
    acc_ref[7] = acc_ref[7] + _l2h(ip_cf)
    acc_ref[8] = acc_ref[8] + _l2h(in_cf)

    @pl.when(i == _NB - 1)
    def _():
        bf = jnp.float32(_B)
        kg_total = acc_ref[0] / bf + _KG_LAMBDA * (
            (acc_ref[1] + acc_ref[2] + acc_ref[3] + acc_ref[4]) / bf)
        cf_total = acc_ref[5] / bf + _CF_LAMBDA * (
            (acc_ref[6] + acc_ref[7] + acc_ref[8]) / bf)
        out_ref[0, 0] = kg_total + cf_total


def _tc_loss(gath, r4, w_big, rel_emb):
    return pl.pallas_call(
        _tc_body,
        grid=(_NB,),
        in_specs=[
            pl.BlockSpec((8, _BLK4, 128), lambda i: (0, i, 0)),
            pl.BlockSpec((_BLK4, 4), lambda i: (i, 0)),
            pl.BlockSpec((_KP, _NP), lambda i: (0, 0)),
            pl.BlockSpec((_NREL, _RD), lambda i: (0, 0)),
        ],
        out_specs=pl.BlockSpec((1, 1), lambda i: (0, 0),
                               memory_space=pltpu.SMEM),
        out_shape=jax.ShapeDtypeStruct((1, 1), jnp.float32),
        scratch_shapes=[pltpu.SMEM((16,), jnp.float32)],
    )(gath, r4, w_big, rel_emb)


def kernel(user_ids, item_pos_ids, item_neg_ids, h, r, pos_t, neg_t,
           user_embed, item_embed, entity_embed, relation_embed, trans_M):
    uid = user_ids.astype(jnp.int32)
    ipi = item_pos_ids.astype(jnp.int32)
    ini = item_neg_ids.astype(jnp.int32)
    h32 = h.astype(jnp.int32)
    pt = pos_t.astype(jnp.int32)
    nt = neg_t.astype(jnp.int32)
    ep = _tc_pack(entity_embed, _N_ENT, 20000)
    up, ipk = _sc_pack(user_embed, item_embed)
    gath = _sc_gather(ep, up, ipk, uid, ipi, ini, h32, pt, nt)
    w_big = trans_M.reshape(_NG, _RG, _D, _RD).transpose(1, 2, 0, 3)
    w_big = w_big.reshape(_KP, _NP)
    r4 = r.astype(jnp.int32).reshape(_B4, 4)
    out = _tc_loss(gath, r4, w_big, relation_embed)
    return out.reshape(())


# speedup vs baseline: 1.1258x; 1.0987x over previous
import functools

import jax
import jax.numpy as jnp
from jax import lax
from jax.experimental import pallas as pl
from jax.experimental.pallas import tpu as pltpu
from jax.experimental.pallas import tpu_sc as plsc

_B = 16384
_B4 = _B // 4
_D = 32
_RD = 32
_NREL = 64
_NW = 32
_BPW = _B // _NW
_NG = 8
_RG = _NREL // _NG
_KP = _RG * _D
_NP = _NG * _RD
_BLK4 = 256
_BLK = 4 * _BLK4
_NB = _B // _BLK
_CF_LAMBDA = 1e-05
_KG_LAMBDA = 1e-05

_N_ENT = 1000000
_N_USR = 1000000
_N_ITM = 100000
_UCH = 320
_UNCH = _N_USR // _UCH
_ICH = 160
_INCH = _N_ITM // _ICH


def _tc_pack(table, n, blkr):
    def body(in_ref, out_ref):
        y = in_ref[...].reshape(blkr // 4, 4, 32)
        out_ref[...] = jnp.concatenate([y[:, q, :] for q in range(4)], axis=1)

    return pl.pallas_call(
        body, grid=(n // blkr,),
        in_specs=[pl.BlockSpec((blkr, 32), lambda i: (i, 0))],
        out_specs=pl.BlockSpec((blkr // 4, 128), lambda i: (i, 0)),
        out_shape=jax.ShapeDtypeStruct((n // 4, 128), jnp.float32))(table)


def _sc_pack_one(tbl, out, ub, ob, sems, osems, w, nch, ch, npack):
    nw2 = (nch + 2 * _NW - 1) // (2 * _NW)
    for par in (0, 1):
        cid0 = w + _NW * par
        @pl.when(cid0 < nch)
        def _(par=par, cid0=cid0):
            pltpu.async_copy(tbl.at[pl.ds(pl.multiple_of(cid0 * ch, 8), ch)],
                             ub[par].at[pl.ds(0, ch)], sems[par])

    def body(c2, _):
        for par in (0, 1):
            c = 2 * c2 + par
            cid = w + _NW * c

            @pl.when(cid < nch)
            def _(par=par, c=c, cid=cid):
                pltpu.make_async_copy(
                    tbl.at[pl.ds(pl.multiple_of(cid * ch, 8), ch)],
                    ub[par].at[pl.ds(0, ch)], sems[par]).wait()

                prev = cid - 2 * _NW

                @pl.when(prev >= 0)
                def _():
                    pltpu.make_async_copy(
                        ob[par].at[pl.ds(0, npack)],
                        out.at[pl.ds(pl.multiple_of(prev * npack, 8), npack)],
                        osems[par]).wait()

                def pack(m, _):
                    for qq in range(4):
                        j = 4 * m + qq
                        ob[par][m, pl.ds(qq * 32, 16)] = ub[par][j, 0:16]
                        ob[par][m, pl.ds(qq * 32 + 16, 16)] = (
                            ub[par][j, pl.ds(16, 16)])
                    return 0

                lax.fori_loop(0, npack, pack, 0, unroll=4)
                nxt = cid + 2 * _NW

                @pl.when(nxt < nch)
                def _():
                    pltpu.async_copy(
                        tbl.at[pl.ds(pl.multiple_of(nxt * ch, 8), ch)],
                        ub[par].at[pl.ds(0, ch)], sems[par])
                pltpu.async_copy(
                    ob[par].at[pl.ds(0, npack)],
                    out.at[pl.ds(pl.multiple_of(cid * npack, 8), npack)],
                    osems[par])
        return 0

    lax.fori_loop(0, nw2, body, 0)
    ncw = (nch - w + _NW - 1) // _NW
    for par in (0, 1):
        @pl.when(ncw > par)
        def _(par=par):
            lastc = ((ncw - 1 - par) // 2) * 2 + par
            lastcid = w + _NW * lastc
            pltpu.make_async_copy(
                ob[par].at[pl.ds(0, npack)],
                out.at[pl.ds(pl.multiple_of(lastcid * npack, 8), npack)],
                osems[par]).wait()


def _sc_pack_kernel(user, item, up, ip, ub0, ub1, ob0, ob1, s0, s1, o0, o1):
    w = lax.axis_index("s") * 2 + lax.axis_index("c")
    _sc_pack_one(user, up, (ub0, ub1), (ob0, ob1), (s0, s1), (o0, o1),
                 w, _UNCH, _UCH, _UCH // 4)
    _sc_pack_one(item, ip, (ub0, ub1), (ob0, ob1), (s0, s1), (o0, o1),
                 w, _INCH, _ICH, _ICH // 4)


def _sc_pack(user_embed, item_embed):
    mesh = plsc.VectorSubcoreMesh(core_axis_name="c", subcore_axis_name="s")
    k = functools.partial(
        pl.kernel,
        mesh=mesh,
        compiler_params=pltpu.CompilerParams(
            needs_layout_passes=False, use_tc_tiling_on_sc=True),
        out_type=(jax.ShapeDtypeStruct((_N_USR // 4, 128), jnp.float32),
                  jax.ShapeDtypeStruct((_N_ITM // 4, 128), jnp.float32)),
        scratch_types=[
            pltpu.VMEM((_UCH, _D), jnp.float32),
            pltpu.VMEM((_UCH, _D), jnp.float32),
            pltpu.VMEM((_UCH // 4, 128), jnp.float32),
            pltpu.VMEM((_UCH // 4, 128), jnp.float32),
            pltpu.SemaphoreType.DMA,
            pltpu.SemaphoreType.DMA,
            pltpu.SemaphoreType.DMA,
            pltpu.SemaphoreType.DMA,
        ],
    )(_sc_pack_kernel)
    return k(user_embed, item_embed)


_GCH = 128


def _sc_gather_kernel(ep, up, ipk, uid_h, ipi_h, ini_h, h_h, pt_h, nt_h, out,
                      iv, tv0, tv1, tb0, tb1, ob, s0, s1, so):
    w = lax.axis_index("s") * 2 + lax.axis_index("c")
    base = pl.multiple_of(w * _BPW, 8)
    sl = pl.ds(base, _BPW)
    tasks = [(ep, h_h), (ep, pt_h), (ep, nt_h), (up, uid_h),
             (ipk, ipi_h), (ipk, ini_h), (ep, ipi_h), (ep, ini_h)]
    tvs = (tv0, tv1)
    tbs = (tb0, tb1)
    sems = (s0, s1)
    owrite = [None]
    for g, (tbl, idx_h) in enumerate(tasks):
        pltpu.sync_copy(idx_h.at[sl], iv)
        for v in range(_GCH // 16):
            tv0[pl.ds(v * 16, 16)] = lax.shift_right_logical(
                iv[pl.ds(v * 16, 16)], 2)
        pltpu.async_copy(tbl.at[tv0], tb0, s0)
        for c in range(_BPW // _GCH):
            par = c % 2
            npar = (c + 1) % 2
            if c + 1 < _BPW // _GCH:
                off = (c + 1) * _GCH
                for v in range(_GCH // 16):
                    tvs[npar][pl.ds(v * 16, 16)] = lax.shift_right_logical(
                        iv[pl.ds(off + v * 16, 16)], 2)
            pltpu.make_async_copy(tbl.at[tvs[par]], tbs[par], sems[par]).wait()
            if c + 1 < _BPW // _GCH:
                pltpu.async_copy(tbl.at[tvs[npar]], tbs[npar], sems[npar])

            ii = lax.iota(jnp.int32, 16)

            def extract(m, _, c=c, par=par):
                for qq in range(4):
                    j = 4 * m + qq
                    jb = jnp.zeros((16,), jnp.int32) + j
                    idxb = plsc.load_gather(iv, [jb + (c * _GCH)])
                    q32 = lax.bitwise_and(idxb, 3) * 32
                    colv = q32 + ii
                    v0 = plsc.load_gather(tbs[par], [jb, colv])
                    v1 = plsc.load_gather(tbs[par], [jb, colv + 16])
                    ob[c * (_GCH // 4) + m, pl.ds(qq * 32, 16)] = v0
                    ob[c * (_GCH // 4) + m, pl.ds(qq * 32 + 16, 16)] = v1
                return 0

            lax.fori_loop(0, _GCH // 4, extract, 0, unroll=2)
        if owrite[0] is not None:
            owrite[0].wait()
        owrite[0] = pltpu.async_copy(
            ob, out.at[g, pl.ds(pl.multiple_of(w * (_BPW // 4), 8),
                                _BPW // 4)], so)
    owrite[0].wait()


def _sc_gather(ep, up, ipk, uid, ipi, ini, h, pt, nt):
    mesh = plsc.VectorSubcoreMesh(core_axis_name="c", subcore_axis_name="s")
    k = functools.partial(
        pl.kernel,
        mesh=mesh,
        compiler_params=pltpu.CompilerParams(
            needs_layout_passes=False, use_tc_tiling_on_sc=True),
        out_type=jax.ShapeDtypeStruct((8, _B4, 128), jnp.float32),
        scratch_types=[
            pltpu.VMEM((_BPW,), jnp.int32),
            pltpu.VMEM((_GCH,), jnp.int32),
            pltpu.VMEM((_GCH,), jnp.int32),
            pltpu.VMEM((_GCH, 128), jnp.float32),
            pltpu.VMEM((_GCH, 128), jnp.float32),
            pltpu.VMEM((_BPW // 4, 128), jnp.float32),
            pltpu.SemaphoreType.DMA,
            pltpu.SemaphoreType.DMA,
            pltpu.SemaphoreType.DMA,
        ],
    )(_sc_gather_kernel)
    return k(ep, up, ipk, uid, ipi, ini, h, pt, nt)


def _logsig(x):
    return jnp.minimum(x, 0.0) - jnp.log1p(jnp.exp(-jnp.abs(x)))


def _nrm(v):
    n = jnp.sqrt(jnp.sum(v * v, axis=1, keepdims=True))
    return v / jnp.maximum(n, 1e-12)


def _l2h(v):
    return 0.5 * jnp.sum(v * v)


def _deint(g_ref, a):
    arr = g_ref[a]
    return [arr[:, q * 32:(q + 1) * 32] for q in range(4)]


def _tc_body(g_ref, r_ref, w_ref, re_ref, out_ref, acc_ref):
    i = pl.program_id(0)

    @pl.when(i == 0)
    def _():
        for k in range(9):
            acc_ref[k] = 0.0

    rcat = jnp.concatenate(
        [r_ref[:, q:q + 1] for q in range(4)], axis=0)
    rl = lax.bitwise_and(rcat, _RG - 1)
    gg = lax.shift_right_logical(rcat, 3)

    h_l = _deint(g_ref, 0)
    pt_l = _deint(g_ref, 1)
    nt_l = _deint(g_ref, 2)

    x3 = jnp.concatenate(h_l + pt_l + nt_l, axis=0)
    xt = pltpu.repeat(x3, _RG, axis=1)
    col = lax.broadcasted_iota(jnp.int32, (3 * _BLK, _KP), 1)
    rlcol = lax.shift_right_logical(col, 5)
    rl3 = jnp.concatenate([rl, rl, rl], axis=0)
    a = jnp.where(rlcol == rl3, xt, 0.0)
    rm_big = jnp.dot(a, w_ref[...], preferred_element_type=jnp.float32)
    gg3 = jnp.concatenate([gg, gg, gg], axis=0)
    rm3 = jnp.zeros((3 * _BLK, _RD), jnp.float32)
    for g in range(_NG):
        rm3 = rm3 + jnp.where(gg3 == g, rm_big[:, g * _RD:(g + 1) * _RD], 0.0)
    rmh = rm3[0:_BLK]
    rmpt = rm3[_BLK:2 * _BLK]
    rmnt = rm3[2 * _BLK:]

    oh = (lax.broadcasted_iota(jnp.int32, (_BLK, _NREL), 1) == rcat)
    r_e = jnp.dot(oh.astype(jnp.float32), re_ref[...],
                  preferred_element_type=jnp.float32)

    rmh_n = _nrm(rmh)
    re_n = _nrm(r_e)
    rmpt_n = _nrm(rmpt)
    rmnt_n = _nrm(rmnt)
    pos_sc = jnp.sum((rmh_n + re_n - rmpt_n) ** 2, axis=1, keepdims=True)
    neg_sc = jnp.sum((rmh_n + re_n - rmnt_n) ** 2, axis=1, keepdims=True)
    kg_ls = jnp.sum(-_logsig(neg_sc - pos_sc))

    u_e = jnp.concatenate(_deint(g_ref, 3), axis=0)
    ip_e = jnp.concatenate(_deint(g_ref, 4), axis=0)
    in_e = jnp.concatenate(_deint(g_ref, 5), axis=0)
    ipk_e = jnp.concatenate(_deint(g_ref, 6), axis=0)
    ink_e = jnp.concatenate(_deint(g_ref, 7), axis=0)
    ip_cf = ip_e + ipk_e
    in_cf = in_e + ink_e
    pos_s = jnp.sum(u_e * ip_cf, axis=1, keepdims=True)
    neg_s = jnp.sum(u_e * in_cf, axis=1, keepdims=True)
    cf_ls = jnp.sum(-_logsig(pos_s - neg_s))

    acc_ref[0] = acc_ref[0] + kg_ls
    acc_ref[1] = acc_ref[1] + _l2h(rmh_n)
    acc_ref[2] = acc_ref[2] + _l2h(re_n)
    acc_ref[3] = acc_ref[3] + _l2h(rmpt_n)
    acc_ref[4] = acc_ref[4] + _l2h(rmnt_n)
    acc_ref[5] = acc_ref[5] + cf_ls
    acc_ref[6] = acc_ref[6] + _l2h(u_e)
    acc_ref[7] = acc_ref[7] + _l2h(ip_cf)
    acc_ref[8] = acc_ref[8] + _l2h(in_cf)

    @pl.when(i == _NB - 1)
    def _():
        bf = jnp.float32(_B)
        kg_total = acc_ref[0] / bf + _KG_LAMBDA * (
            (acc_ref[1] + acc_ref[2] + acc_ref[3] + acc_ref[4]) / bf)
        cf_total = acc_ref[5] / bf + _CF_LAMBDA * (
            (acc_ref[6] + acc_ref[7] + acc_ref[8]) / bf)
        out_ref[0, 0] = kg_total + cf_total


def _tc_loss(gath, r4, w_big, rel_emb):
    return pl.pallas_call(
        _tc_body,
        grid=(_NB,),
        in_specs=[
            pl.BlockSpec((8, _BLK4, 128), lambda i: (0, i, 0)),
            pl.BlockSpec((_BLK4, 4), lambda i: (i, 0)),
            pl.BlockSpec((_KP, _NP), lambda i: (0, 0)),
            pl.BlockSpec((_NREL, _RD), lambda i: (0, 0)),
        ],
        out_specs=pl.BlockSpec((1, 1), lambda i: (0, 0),
                               memory_space=pltpu.SMEM),
        out_shape=jax.ShapeDtypeStruct((1, 1), jnp.float32),
        scratch_shapes=[pltpu.SMEM((16,), jnp.float32)],
    )(gath, r4, w_big, rel_emb)


def kernel(user_ids, item_pos_ids, item_neg_ids, h, r, pos_t, neg_t,
           user_embed, item_embed, entity_embed, relation_embed, trans_M):
    uid = user_ids.astype(jnp.int32)
    ipi = item_pos_ids.astype(jnp.int32)
    ini = item_neg_ids.astype(jnp.int32)
    h32 = h.astype(jnp.int32)
    pt = pos_t.astype(jnp.int32)
    nt = neg_t.astype(jnp.int32)
    ep = entity_embed.reshape(_N_ENT // 4, 128)
    up = user_embed.reshape(_N_USR // 4, 128)
    ipk = item_embed.reshape(_N_ITM // 4, 128)
    gath = _sc_gather(ep, up, ipk, uid, ipi, ini, h32, pt, nt)
    w_big = trans_M.reshape(_NG, _RG, _D, _RD).transpose(1, 2, 0, 3)
    w_big = w_big.reshape(_KP, _NP)
    r4 = r.astype(jnp.int32).reshape(_B4, 4)
    out = _tc_loss(gath, r4, w_big, relation_embed)
    return out.reshape(())


# speedup vs baseline: 1.2241x; 1.0873x over previous
import functools

import jax
import jax.numpy as jnp
from jax import lax
from jax.experimental import pallas as pl
from jax.experimental.pallas import tpu as pltpu
from jax.experimental.pallas import tpu_sc as plsc

_B = 16384
_B4 = _B // 4
_D = 32
_RD = 32
_NREL = 64
_NW = 32
_BPW = _B // _NW
_NG = 8
_RG = _NREL // _NG
_KP = _RG * _D
_NP = _NG * _RD
_BLK4 = 256
_BLK = 4 * _BLK4
_NB = _B // _BLK
_CF_LAMBDA = 1e-05
_KG_LAMBDA = 1e-05

_N_ENT = 1000000
_N_USR = 1000000
_N_ITM = 100000
_UCH = 320
_UNCH = _N_USR // _UCH
_ICH = 160
_INCH = _N_ITM // _ICH


def _tc_pack(table, n, blkr):
    def body(in_ref, out_ref):
        y = in_ref[...].reshape(blkr // 4, 4, 32)
        out_ref[...] = jnp.concatenate([y[:, q, :] for q in range(4)], axis=1)

    return pl.pallas_call(
        body, grid=(n // blkr,),
        in_specs=[pl.BlockSpec((blkr, 32), lambda i: (i, 0))],
        out_specs=pl.BlockSpec((blkr // 4, 128), lambda i: (i, 0)),
        out_shape=jax.ShapeDtypeStruct((n // 4, 128), jnp.float32))(table)


def _sc_pack_one(tbl, out, ub, ob, sems, osems, w, nch, ch, npack):
    nw2 = (nch + 2 * _NW - 1) // (2 * _NW)
    for par in (0, 1):
        cid0 = w + _NW * par
        @pl.when(cid0 < nch)
        def _(par=par, cid0=cid0):
            pltpu.async_copy(tbl.at[pl.ds(pl.multiple_of(cid0 * ch, 8), ch)],
                             ub[par].at[pl.ds(0, ch)], sems[par])

    def body(c2, _):
        for par in (0, 1):
            c = 2 * c2 + par
            cid = w + _NW * c

            @pl.when(cid < nch)
            def _(par=par, c=c, cid=cid):
                pltpu.make_async_copy(
                    tbl.at[pl.ds(pl.multiple_of(cid * ch, 8), ch)],
                    ub[par].at[pl.ds(0, ch)], sems[par]).wait()

                prev = cid - 2 * _NW

                @pl.when(prev >= 0)
                def _():
                    pltpu.make_async_copy(
                        ob[par].at[pl.ds(0, npack)],
                        out.at[pl.ds(pl.multiple_of(prev * npack, 8), npack)],
                        osems[par]).wait()

                def pack(m, _):
                    for qq in range(4):
                        j = 4 * m + qq
                        ob[par][m, pl.ds(qq * 32, 16)] = ub[par][j, 0:16]
                        ob[par][m, pl.ds(qq * 32 + 16, 16)] = (
                            ub[par][j, pl.ds(16, 16)])
                    return 0

                lax.fori_loop(0, npack, pack, 0, unroll=4)
                nxt = cid + 2 * _NW

                @pl.when(nxt < nch)
                def _():
                    pltpu.async_copy(
                        tbl.at[pl.ds(pl.multiple_of(nxt * ch, 8), ch)],
                        ub[par].at[pl.ds(0, ch)], sems[par])
                pltpu.async_copy(
                    ob[par].at[pl.ds(0, npack)],
                    out.at[pl.ds(pl.multiple_of(cid * npack, 8), npack)],
                    osems[par])
        return 0

    lax.fori_loop(0, nw2, body, 0)
    ncw = (nch - w + _NW - 1) // _NW
    for par in (0, 1):
        @pl.when(ncw > par)
        def _(par=par):
            lastc = ((ncw - 1 - par) // 2) * 2 + par
            lastcid = w + _NW * lastc
            pltpu.make_async_copy(
                ob[par].at[pl.ds(0, npack)],
                out.at[pl.ds(pl.multiple_of(lastcid * npack, 8), npack)],
                osems[par]).wait()


def _sc_pack_kernel(user, item, up, ip, ub0, ub1, ob0, ob1, s0, s1, o0, o1):
    w = lax.axis_index("s") * 2 + lax.axis_index("c")
    _sc_pack_one(user, up, (ub0, ub1), (ob0, ob1), (s0, s1), (o0, o1),
                 w, _UNCH, _UCH, _UCH // 4)
    _sc_pack_one(item, ip, (ub0, ub1), (ob0, ob1), (s0, s1), (o0, o1),
                 w, _INCH, _ICH, _ICH // 4)


def _sc_pack(user_embed, item_embed):
    mesh = plsc.VectorSubcoreMesh(core_axis_name="c", subcore_axis_name="s")
    k = functools.partial(
        pl.kernel,
        mesh=mesh,
        compiler_params=pltpu.CompilerParams(
            needs_layout_passes=False, use_tc_tiling_on_sc=True),
        out_type=(jax.ShapeDtypeStruct((_N_USR // 4, 128), jnp.float32),
                  jax.ShapeDtypeStruct((_N_ITM // 4, 128), jnp.float32)),
        scratch_types=[
            pltpu.VMEM((_UCH, _D), jnp.float32),
            pltpu.VMEM((_UCH, _D), jnp.float32),
            pltpu.VMEM((_UCH // 4, 128), jnp.float32),
            pltpu.VMEM((_UCH // 4, 128), jnp.float32),
            pltpu.SemaphoreType.DMA,
            pltpu.SemaphoreType.DMA,
            pltpu.SemaphoreType.DMA,
            pltpu.SemaphoreType.DMA,
        ],
    )(_sc_pack_kernel)
    return k(user_embed, item_embed)


_GCH = 128


def _sc_gather_kernel(tasks, out, iv, tv0, tv1, tb0, tb1, ob, s0, s1, so):
    w = lax.axis_index("s") * 2 + lax.axis_index("c")
    base = pl.multiple_of(w * _BPW, 8)
    sl = pl.ds(base, _BPW)
    tvs = (tv0, tv1)
    tbs = (tb0, tb1)
    sems = (s0, s1)
    owrite = [None]
    for g, (tbl, idx_h) in enumerate(tasks):
        pltpu.sync_copy(idx_h.at[sl], iv)
        for v in range(_GCH // 16):
            tv0[pl.ds(v * 16, 16)] = lax.shift_right_logical(
                iv[pl.ds(v * 16, 16)], 2)
        pltpu.async_copy(tbl.at[tv0], tb0, s0)
        for c in range(_BPW // _GCH):
            par = c % 2
            npar = (c + 1) % 2
            if c + 1 < _BPW // _GCH:
                off = (c + 1) * _GCH
                for v in range(_GCH // 16):
                    tvs[npar][pl.ds(v * 16, 16)] = lax.shift_right_logical(
                        iv[pl.ds(off + v * 16, 16)], 2)
            pltpu.make_async_copy(tbl.at[tvs[par]], tbs[par], sems[par]).wait()
            if c + 1 < _BPW // _GCH:
                pltpu.async_copy(tbl.at[tvs[npar]], tbs[npar], sems[npar])

            ii = lax.iota(jnp.int32, 16)

            def extract(m, _, c=c, par=par):
                for qq in range(4):
                    j = 4 * m + qq
                    jb = jnp.zeros((16,), jnp.int32) + j
                    idxb = plsc.load_gather(iv, [jb + (c * _GCH)])
                    q32 = lax.bitwise_and(idxb, 3) * 32
                    colv = q32 + ii
                    v0 = plsc.load_gather(tbs[par], [jb, colv])
                    v1 = plsc.load_gather(tbs[par], [jb, colv + 16])
                    ob[c * (_GCH // 4) + m, pl.ds(qq * 32, 16)] = v0
                    ob[c * (_GCH // 4) + m, pl.ds(qq * 32 + 16, 16)] = v1
                return 0

            lax.fori_loop(0, _GCH // 4, extract, 0, unroll=2)
        if owrite[0] is not None:
            owrite[0].wait()
        owrite[0] = pltpu.async_copy(
            ob, out.at[g, pl.ds(pl.multiple_of(w * (_BPW // 4), 8),
                                _BPW // 4)], so)
    owrite[0].wait()


def _sc_gather(tables, idxs, ngath):
    mesh = plsc.VectorSubcoreMesh(core_axis_name="c", subcore_axis_name="s")

    def body(*refs):
        ntbl = len(tables)
        tbls = refs[:ntbl]
        idx_refs = refs[ntbl:ntbl + len(idxs)]
        out = refs[ntbl + len(idxs)]
        scratch = refs[ntbl + len(idxs) + 1:]
        tasks = [(tbls[ti], idx_refs[ii]) for ti, ii in ngath]
        _sc_gather_kernel(tasks, out, *scratch)

    k = functools.partial(
        pl.kernel,
        mesh=mesh,
        compiler_params=pltpu.CompilerParams(
            needs_layout_passes=False, use_tc_tiling_on_sc=True),
        out_type=jax.ShapeDtypeStruct((len(ngath), _B4, 128), jnp.float32),
        scratch_types=[
            pltpu.VMEM((_BPW,), jnp.int32),
            pltpu.VMEM((_GCH,), jnp.int32),
            pltpu.VMEM((_GCH,), jnp.int32),
            pltpu.VMEM((_GCH, 128), jnp.float32),
            pltpu.VMEM((_GCH, 128), jnp.float32),
            pltpu.VMEM((_BPW // 4, 128), jnp.float32),
            pltpu.SemaphoreType.DMA,
            pltpu.SemaphoreType.DMA,
            pltpu.SemaphoreType.DMA,
        ],
    )(body)
    return k(*tables, *idxs)


def _logsig(x):
    return jnp.minimum(x, 0.0) - jnp.log1p(jnp.exp(-jnp.abs(x)))


def _nrm(v):
    n = jnp.sqrt(jnp.sum(v * v, axis=1, keepdims=True))
    return v / jnp.maximum(n, 1e-12)


def _l2h(v):
    return 0.5 * jnp.sum(v * v)


def _deint(g_ref, a):
    arr = g_ref[a]
    return [arr[:, q * 32:(q + 1) * 32] for q in range(4)]


def _tc_body(ga_ref, gb_ref, r_ref, w_ref, p_ref, re_ref, out_ref, acc_ref):
    i = pl.program_id(0)

    @pl.when(i == 0)
    def _():
        for k in range(9):
            acc_ref[k] = 0.0

    rcat = jnp.concatenate(
        [r_ref[:, q:q + 1] for q in range(4)], axis=0)
    rl = lax.bitwise_and(rcat, _RG - 1)
    gg = lax.shift_right_logical(rcat, 3)

    h_l = _deint(ga_ref, 0)
    pt_l = _deint(ga_ref, 1)
    nt_l = _deint(ga_ref, 2)

    x3 = jnp.concatenate(h_l + pt_l + nt_l, axis=0)
    xt = pltpu.repeat(x3.astype(jnp.bfloat16), _RG, axis=1)
    col = lax.broadcasted_iota(jnp.int32, (3 * _BLK, _KP), 1)
    rlcol = lax.shift_right_logical(col, 5)
    rl3 = jnp.concatenate([rl, rl, rl], axis=0)
    gg3 = jnp.concatenate([gg, gg, gg], axis=0)
    a = jnp.where(rlcol == rl3, xt, jnp.bfloat16(0))
    rm_big = jnp.dot(a, w_ref[...], preferred_element_type=jnp.float32)
    mg = jnp.where(rlcol == gg3, rm_big, 0.0)
    rm3 = jnp.dot(mg, p_ref[...], preferred_element_type=jnp.float32)
    rmh = rm3[0:_BLK]
    rmpt = rm3[_BLK:2 * _BLK]
    rmnt = rm3[2 * _BLK:]

    oh = (lax.broadcasted_iota(jnp.int32, (_BLK, _NREL), 1) == rcat)
    r_e = jnp.dot(oh.astype(jnp.float32), re_ref[...],
                  preferred_element_type=jnp.float32)

    rmh_n = _nrm(rmh)
    re_n = _nrm(r_e)
    rmpt_n = _nrm(rmpt)
    rmnt_n = _nrm(rmnt)
    pos_sc = jnp.sum((rmh_n + re_n - rmpt_n) ** 2, axis=1, keepdims=True)
    neg_sc = jnp.sum((rmh_n + re_n - rmnt_n) ** 2, axis=1, keepdims=True)
    kg_ls = jnp.sum(-_logsig(neg_sc - pos_sc))

    u_e = jnp.concatenate(_deint(gb_ref, 0), axis=0)
    ip_e = jnp.concatenate(_deint(gb_ref, 1), axis=0)
    in_e = jnp.concatenate(_deint(gb_ref, 2), axis=0)
    ipk_e = jnp.concatenate(_deint(ga_ref, 3), axis=0)
    ink_e = jnp.concatenate(_deint(ga_ref, 4), axis=0)
    ip_cf = ip_e + ipk_e
    in_cf = in_e + ink_e
    pos_s = jnp.sum(u_e * ip_cf, axis=1, keepdims=True)
    neg_s = jnp.sum(u_e * in_cf, axis=1, keepdims=True)
    cf_ls = jnp.sum(-_logsig(pos_s - neg_s))

    acc_ref[0] = acc_ref[0] + kg_ls
    acc_ref[1] = acc_ref[1] + _l2h(rmh_n)
    acc_ref[2] = acc_ref[2] + _l2h(re_n)
    acc_ref[3] = acc_ref[3] + _l2h(rmpt_n)
    acc_ref[4] = acc_ref[4] + _l2h(rmnt_n)
    acc_ref[5] = acc_ref[5] + cf_ls
    acc_ref[6] = acc_ref[6] + _l2h(u_e)
    acc_ref[7] = acc_ref[7] + _l2h(ip_cf)
    acc_ref[8] = acc_ref[8] + _l2h(in_cf)

    @pl.when(i == _NB - 1)
    def _():
        bf = jnp.float32(_B)
        kg_total = acc_ref[0] / bf + _KG_LAMBDA * (
            (acc_ref[1] + acc_ref[2] + acc_ref[3] + acc_ref[4]) / bf)
        cf_total = acc_ref[5] / bf + _CF_LAMBDA * (
            (acc_ref[6] + acc_ref[7] + acc_ref[8]) / bf)
        out_ref[0, 0] = kg_total + cf_total


def _tc_loss(ga, gb, r4, w_big, pmat, rel_emb):
    return pl.pallas_call(
        _tc_body,
        grid=(_NB,),
        in_specs=[
            pl.BlockSpec((5, _BLK4, 128), lambda i: (0, i, 0)),
            pl.BlockSpec((3, _BLK4, 128), lambda i: (0, i, 0)),
            pl.BlockSpec((_BLK4, 4), lambda i: (i, 0)),
            pl.BlockSpec((_KP, _NP), lambda i: (0, 0)),
            pl.BlockSpec((_NP, _RD), lambda i: (0, 0)),
            pl.BlockSpec((_NREL, _RD), lambda i: (0, 0)),
        ],
        out_specs=pl.BlockSpec((1, 1), lambda i: (0, 0),
                               memory_space=pltpu.SMEM),
        out_shape=jax.ShapeDtypeStruct((1, 1), jnp.float32),
        scratch_shapes=[pltpu.SMEM((16,), jnp.float32)],
    )(ga, gb, r4, w_big, pmat, rel_emb)


def kernel(user_ids, item_pos_ids, item_neg_ids, h, r, pos_t, neg_t,
           user_embed, item_embed, entity_embed, relation_embed, trans_M):
    uid = user_ids.astype(jnp.int32)
    ipi = item_pos_ids.astype(jnp.int32)
    ini = item_neg_ids.astype(jnp.int32)
    h32 = h.astype(jnp.int32)
    pt = pos_t.astype(jnp.int32)
    nt = neg_t.astype(jnp.int32)
    up = user_embed.reshape(_N_USR // 4, 128)
    ipp = item_embed.reshape(_N_ITM // 4, 128)
    gb = _sc_gather((up, ipp), (uid, ipi, ini),
                    [(0, 0), (1, 1), (1, 2)])
    ep = entity_embed.reshape(_N_ENT // 4, 128)
    ga = _sc_gather((ep,), (h32, pt, nt, ipi, ini),
                    [(0, 0), (0, 1), (0, 2), (0, 3), (0, 4)])
    w_big = trans_M.reshape(_NG, _RG, _D, _RD).transpose(1, 2, 0, 3)
    w_big = w_big.reshape(_KP, _NP).astype(jnp.bfloat16)
    pmat = (lax.broadcasted_iota(jnp.int32, (_NP, _RD), 0) % _RD ==
            lax.broadcasted_iota(jnp.int32, (_NP, _RD), 1)).astype(jnp.float32)
    r4 = r.astype(jnp.int32).reshape(_B4, 4)
    out = _tc_loss(ga, gb, r4, w_big, pmat, relation_embed)
    return out.reshape(())


# speedup vs baseline: 1.3044x; 1.0655x over previous
import functools

import jax
import jax.numpy as jnp
from jax import lax
from jax.experimental import pallas as pl
from jax.experimental.pallas import tpu as pltpu
from jax.experimental.pallas import tpu_sc as plsc

_B = 16384
_B4 = _B // 4
_D = 32
_RD = 32
_NREL = 64
_NW = 32
_BPW = _B // _NW
_NG = 8
_RG = _NREL // _NG
_KP = _RG * _D
_NP = _NG * _RD
_BLK4 = 256
_BLK = 4 * _BLK4
_NB = _B // _BLK
_CF_LAMBDA = 1e-05
_KG_LAMBDA = 1e-05

_N_ENT = 1000000
_N_USR = 1000000
_N_ITM = 100000
_UCH = 320
_UNCH = _N_USR // _UCH
_ICH = 160
_INCH = _N_ITM // _ICH


def _tc_pack(table, n, blkr):
    def body(in_ref, out_ref):
        y = in_ref[...].reshape(blkr // 4, 4, 32)
        out_ref[...] = jnp.concatenate([y[:, q, :] for q in range(4)], axis=1)

    return pl.pallas_call(
        body, grid=(n // blkr,),
        in_specs=[pl.BlockSpec((blkr, 32), lambda i: (i, 0))],
        out_specs=pl.BlockSpec((blkr // 4, 128), lambda i: (i, 0)),
        out_shape=jax.ShapeDtypeStruct((n // 4, 128), jnp.float32))(table)


def _sc_pack_one(tbl, out, ub, ob, sems, osems, w, nch, ch, npack):
    nw2 = (nch + 2 * _NW - 1) // (2 * _NW)
    for par in (0, 1):
        cid0 = w + _NW * par
        @pl.when(cid0 < nch)
        def _(par=par, cid0=cid0):
            pltpu.async_copy(tbl.at[pl.ds(pl.multiple_of(cid0 * ch, 8), ch)],
                             ub[par].at[pl.ds(0, ch)], sems[par])

    def body(c2, _):
        for par in (0, 1):
            c = 2 * c2 + par
            cid = w + _NW * c

            @pl.when(cid < nch)
            def _(par=par, c=c, cid=cid):
                pltpu.make_async_copy(
                    tbl.at[pl.ds(pl.multiple_of(cid * ch, 8), ch)],
                    ub[par].at[pl.ds(0, ch)], sems[par]).wait()

                prev = cid - 2 * _NW

                @pl.when(prev >= 0)
                def _():
                    pltpu.make_async_copy(
                        ob[par].at[pl.ds(0, npack)],
                        out.at[pl.ds(pl.multiple_of(prev * npack, 8), npack)],
                        osems[par]).wait()

                def pack(m, _):
                    for qq in range(4):
                        j = 4 * m + qq
                        ob[par][m, pl.ds(qq * 32, 16)] = ub[par][j, 0:16]
                        ob[par][m, pl.ds(qq * 32 + 16, 16)] = (
                            ub[par][j, pl.ds(16, 16)])
                    return 0

                lax.fori_loop(0, npack, pack, 0, unroll=4)
                nxt = cid + 2 * _NW

                @pl.when(nxt < nch)
                def _():
                    pltpu.async_copy(
                        tbl.at[pl.ds(pl.multiple_of(nxt * ch, 8), ch)],
                        ub[par].at[pl.ds(0, ch)], sems[par])
                pltpu.async_copy(
                    ob[par].at[pl.ds(0, npack)],
                    out.at[pl.ds(pl.multiple_of(cid * npack, 8), npack)],
                    osems[par])
        return 0

    lax.fori_loop(0, nw2, body, 0)
    ncw = (nch - w + _NW - 1) // _NW
    for par in (0, 1):
        @pl.when(ncw > par)
        def _(par=par):
            lastc = ((ncw - 1 - par) // 2) * 2 + par
            lastcid = w + _NW * lastc
            pltpu.make_async_copy(
                ob[par].at[pl.ds(0, npack)],
                out.at[pl.ds(pl.multiple_of(lastcid * npack, 8), npack)],
                osems[par]).wait()


def _sc_pack_kernel(user, up, ub0, ub1, ob0, ob1, s0, s1, o0, o1):
    w = lax.axis_index("s") * 2 + lax.axis_index("c")
    _sc_pack_one(user, up, (ub0, ub1), (ob0, ob1), (s0, s1), (o0, o1),
                 w, _UNCH, _UCH, _UCH // 4)


def _sc_pack(user_embed):
    mesh = plsc.VectorSubcoreMesh(core_axis_name="c", subcore_axis_name="s")
    k = functools.partial(
        pl.kernel,
        mesh=mesh,
        compiler_params=pltpu.CompilerParams(
            needs_layout_passes=False, use_tc_tiling_on_sc=True),
        out_type=jax.ShapeDtypeStruct((_N_USR // 4, 128), jnp.float32),
        scratch_types=[
            pltpu.VMEM((_UCH, _D), jnp.float32),
            pltpu.VMEM((_UCH, _D), jnp.float32),
            pltpu.VMEM((_UCH // 4, 128), jnp.float32),
            pltpu.VMEM((_UCH // 4, 128), jnp.float32),
            pltpu.SemaphoreType.DMA,
            pltpu.SemaphoreType.DMA,
            pltpu.SemaphoreType.DMA,
            pltpu.SemaphoreType.DMA,
        ],
    )(_sc_pack_kernel)
    return k(user_embed)


_GCH = 128


def _sc_gather_kernel(tasks, out, iv, tv0, tv1, tb0, tb1, ob, s0, s1, so):
    w = lax.axis_index("s") * 2 + lax.axis_index("c")
    base = pl.multiple_of(w * _BPW, 8)
    sl = pl.ds(base, _BPW)
    tvs = (tv0, tv1)
    tbs = (tb0, tb1)
    sems = (s0, s1)
    owrite = [None]
    for g, (tbl, idx_h) in enumerate(tasks):
        pltpu.sync_copy(idx_h.at[sl], iv)
        for v in range(_GCH // 16):
            tv0[pl.ds(v * 16, 16)] = lax.shift_right_logical(
                iv[pl.ds(v * 16, 16)], 2)
        pltpu.async_copy(tbl.at[tv0], tb0, s0)
        for c in range(_BPW // _GCH):
            par = c % 2
            npar = (c + 1) % 2
            if c + 1 < _BPW // _GCH:
                off = (c + 1) * _GCH
                for v in range(_GCH // 16):
                    tvs[npar][pl.ds(v * 16, 16)] = lax.shift_right_logical(
                        iv[pl.ds(off + v * 16, 16)], 2)
            pltpu.make_async_copy(tbl.at[tvs[par]], tbs[par], sems[par]).wait()
            if c + 1 < _BPW // _GCH:
                pltpu.async_copy(tbl.at[tvs[npar]], tbs[npar], sems[npar])

            ii = lax.iota(jnp.int32, 16)

            def extract(m, _, c=c, par=par):
                for qq in range(4):
                    j = 4 * m + qq
                    jb = jnp.zeros((16,), jnp.int32) + j
                    idxb = plsc.load_gather(iv, [jb + (c * _GCH)])
                    q32 = lax.bitwise_and(idxb, 3) * 32
                    colv = q32 + ii
                    v0 = plsc.load_gather(tbs[par], [jb, colv])
                    v1 = plsc.load_gather(tbs[par], [jb, colv + 16])
                    ob[c * (_GCH // 4) + m, pl.ds(qq * 32, 16)] = v0
                    ob[c * (_GCH // 4) + m, pl.ds(qq * 32 + 16, 16)] = v1
                return 0

            lax.fori_loop(0, _GCH // 4, extract, 0, unroll=2)
        if owrite[0] is not None:
            owrite[0].wait()
        owrite[0] = pltpu.async_copy(
            ob, out.at[g, pl.ds(pl.multiple_of(w * (_BPW // 4), 8),
                                _BPW // 4)], so)
    owrite[0].wait()


def _sc_gather(tables, idxs, ngath):
    mesh = plsc.VectorSubcoreMesh(core_axis_name="c", subcore_axis_name="s")

    def body(*refs):
        ntbl = len(tables)
        tbls = refs[:ntbl]
        idx_refs = refs[ntbl:ntbl + len(idxs)]
        out = refs[ntbl + len(idxs)]
        scratch = refs[ntbl + len(idxs) + 1:]
        tasks = [(tbls[ti], idx_refs[ii]) for ti, ii in ngath]
        _sc_gather_kernel(tasks, out, *scratch)

    k = functools.partial(
        pl.kernel,
        mesh=mesh,
        compiler_params=pltpu.CompilerParams(
            needs_layout_passes=False, use_tc_tiling_on_sc=True),
        out_type=jax.ShapeDtypeStruct((len(ngath), _B4, 128), jnp.float32),
        scratch_types=[
            pltpu.VMEM((_BPW,), jnp.int32),
            pltpu.VMEM((_GCH,), jnp.int32),
            pltpu.VMEM((_GCH,), jnp.int32),
            pltpu.VMEM((_GCH, 128), jnp.float32),
            pltpu.VMEM((_GCH, 128), jnp.float32),
            pltpu.VMEM((_BPW // 4, 128), jnp.float32),
            pltpu.SemaphoreType.DMA,
            pltpu.SemaphoreType.DMA,
            pltpu.SemaphoreType.DMA,
        ],
    )(body)
    return k(*tables, *idxs)


def _logsig(x):
    return jnp.minimum(x, 0.0) - jnp.log1p(jnp.exp(-jnp.abs(x)))


def _nrm(v):
    n = jnp.sqrt(jnp.sum(v * v, axis=1, keepdims=True))
    return v / jnp.maximum(n, 1e-12)


def _l2h(v):
    return 0.5 * jnp.sum(v * v)


def _deint(g_ref, a):
    arr = g_ref[a]
    return [arr[:, q * 32:(q + 1) * 32] for q in range(4)]


def _tc_body(ga_ref, gb_ref, r_ref, w_ref, p_ref, re_ref, out_ref, acc_ref):
    i = pl.program_id(0)

    @pl.when(i == 0)
    def _():
        for k in range(9):
            acc_ref[k] = 0.0

    rcat = jnp.concatenate(
        [r_ref[:, q:q + 1] for q in range(4)], axis=0)
    rl = lax.bitwise_and(rcat, _RG - 1)
    gg = lax.shift_right_logical(rcat, 3)

    h_l = _deint(ga_ref, 0)
    pt_l = _deint(ga_ref, 1)
    nt_l = _deint(ga_ref, 2)

    x3 = jnp.concatenate(h_l + pt_l + nt_l, axis=0)
    xt = pltpu.repeat(x3.astype(jnp.bfloat16), _RG, axis=1)
    col = lax.broadcasted_iota(jnp.int32, (3 * _BLK, _KP), 1)
    rlcol = lax.shift_right_logical(col, 5)
    rl3 = jnp.concatenate([rl, rl, rl], axis=0)
    gg3 = jnp.concatenate([gg, gg, gg], axis=0)
    a = jnp.where(rlcol == rl3, xt, jnp.bfloat16(0))
    rm_big = jnp.dot(a, w_ref[...], preferred_element_type=jnp.float32)
    mg = jnp.where(rlcol == gg3, rm_big, 0.0)
    rm3 = jnp.dot(mg, p_ref[...], preferred_element_type=jnp.float32)
    rmh = rm3[0:_BLK]
    rmpt = rm3[_BLK:2 * _BLK]
    rmnt = rm3[2 * _BLK:]

    oh = (lax.broadcasted_iota(jnp.int32, (_BLK, _NREL), 1) == rcat)
    r_e = jnp.dot(oh.astype(jnp.float32), re_ref[...],
                  preferred_element_type=jnp.float32)

    rmh_n = _nrm(rmh)
    re_n = _nrm(r_e)
    rmpt_n = _nrm(rmpt)
    rmnt_n = _nrm(rmnt)
    pos_sc = jnp.sum((rmh_n + re_n - rmpt_n) ** 2, axis=1, keepdims=True)
    neg_sc = jnp.sum((rmh_n + re_n - rmnt_n) ** 2, axis=1, keepdims=True)
    kg_ls = jnp.sum(-_logsig(neg_sc - pos_sc))

    u_e = jnp.concatenate(_deint(gb_ref, 0), axis=0)
    ip_e = jnp.concatenate(_deint(gb_ref, 1), axis=0)
    in_e = jnp.concatenate(_deint(gb_ref, 2), axis=0)
    ipk_e = jnp.concatenate(_deint(ga_ref, 3), axis=0)
    ink_e = jnp.concatenate(_deint(ga_ref, 4), axis=0)
    ip_cf = ip_e + ipk_e
    in_cf = in_e + ink_e
    pos_s = jnp.sum(u_e * ip_cf, axis=1, keepdims=True)
    neg_s = jnp.sum(u_e * in_cf, axis=1, keepdims=True)
    cf_ls = jnp.sum(-_logsig(pos_s - neg_s))

    acc_ref[0] = acc_ref[0] + kg_ls
    acc_ref[1] = acc_ref[1] + _l2h(rmh_n)
    acc_ref[2] = acc_ref[2] + _l2h(re_n)
    acc_ref[3] = acc_ref[3] + _l2h(rmpt_n)
    acc_ref[4] = acc_ref[4] + _l2h(rmnt_n)
    acc_ref[5] = acc_ref[5] + cf_ls
    acc_ref[6] = acc_ref[6] + _l2h(u_e)
    acc_ref[7] = acc_ref[7] + _l2h(ip_cf)
    acc_ref[8] = acc_ref[8] + _l2h(in_cf)

    @pl.when(i == _NB - 1)
    def _():
        bf = jnp.float32(_B)
        kg_total = acc_ref[0] / bf + _KG_LAMBDA * (
            (acc_ref[1] + acc_ref[2] + acc_ref[3] + acc_ref[4]) / bf)
        cf_total = acc_ref[5] / bf + _CF_LAMBDA * (
            (acc_ref[6] + acc_ref[7] + acc_ref[8]) / bf)
        out_ref[0, 0] = kg_total + cf_total


def _tc_loss(ga, gb, r4, w_big, pmat, rel_emb):
    return pl.pallas_call(
        _tc_body,
        grid=(_NB,),
        in_specs=[
            pl.BlockSpec((5, _BLK4, 128), lambda i: (0, i, 0)),
            pl.BlockSpec((3, _BLK4, 128), lambda i: (0, i, 0)),
            pl.BlockSpec((_BLK4, 4), lambda i: (i, 0)),
            pl.BlockSpec((_KP, _NP), lambda i: (0, 0)),
            pl.BlockSpec((_NP, _RD), lambda i: (0, 0)),
            pl.BlockSpec((_NREL, _RD), lambda i: (0, 0)),
        ],
        out_specs=pl.BlockSpec((1, 1), lambda i: (0, 0),
                               memory_space=pltpu.SMEM),
        out_shape=jax.ShapeDtypeStruct((1, 1), jnp.float32),
        scratch_shapes=[pltpu.SMEM((16,), jnp.float32)],
    )(ga, gb, r4, w_big, pmat, rel_emb)


def kernel(user_ids, item_pos_ids, item_neg_ids, h, r, pos_t, neg_t,
           user_embed, item_embed, entity_embed, relation_embed, trans_M):
    uid = user_ids.astype(jnp.int32)
    ipi = item_pos_ids.astype(jnp.int32)
    ini = item_neg_ids.astype(jnp.int32)
    h32 = h.astype(jnp.int32)
    pt = pos_t.astype(jnp.int32)
    nt = neg_t.astype(jnp.int32)
    up = _sc_pack(user_embed)
    ipp = item_embed.reshape(_N_ITM // 4, 128)
    gb = _sc_gather((up, ipp), (uid, ipi, ini),
                    [(0, 0), (1, 1), (1, 2)])
    ep = entity_embed.reshape(_N_ENT // 4, 128)
    ga = _sc_gather((ep,), (h32, pt, nt, ipi, ini),
                    [(0, 0), (0, 1), (0, 2), (0, 3), (0, 4)])
    w_big = trans_M.reshape(_NG, _RG, _D, _RD).transpose(1, 2, 0, 3)
    w_big = w_big.reshape(_KP, _NP).astype(jnp.bfloat16)
    pmat = (lax.broadcasted_iota(jnp.int32, (_NP, _RD), 0) % _RD ==
            lax.broadcasted_iota(jnp.int32, (_NP, _RD), 1)).astype(jnp.float32)
    r4 = r.astype(jnp.int32).reshape(_B4, 4)
    out = _tc_loss(ga, gb, r4, w_big, pmat, relation_embed)
    return out.reshape(())
